# baseline (device time: 61239 ns/iter reference)
import jax
import jax.numpy as jnp
from jax import lax
from jax.experimental import pallas as pl
from jax.experimental.pallas import tpu as pltpu

N_DEV = 4
B, SQ, HQ, HKV, DH = 4, 256, 8, 2, 128
GQ = HQ // HKV
SKV_LOC = 1024
D = HQ * DH
R = GQ * SQ
NBG = B * HKV
SCALE = 0.08838834764831843

BROWS = R + 8


def kernel(x, Wq, Wo, K_ext, V_ext):

    def body(x_ref, wq_ref, wo_ref, k_ref, v_ref, out_ref,
             gbuf, send_sems, recv_sems):
        my = lax.axis_index("i")
        left = (my + N_DEV - 1) % N_DEV
        right = (my + 1) % N_DEV

        bsem = pltpu.get_barrier_semaphore()
        for nbr in (left, right):
            pl.semaphore_signal(
                bsem, inc=1, device_id=(nbr,),
                device_id_type=pl.DeviceIdType.MESH,
            )
        pl.semaphore_wait(bsem, 2)

        def xfer(src_slot, dst_slot, j, target):
            return pltpu.make_async_remote_copy(
                src_ref=gbuf.at[src_slot, j],
                dst_ref=gbuf.at[dst_slot, j],
                send_sem=send_sems.at[dst_slot, j],
                recv_sem=recv_sems.at[dst_slot, j],
                device_id=(target,), device_id_type=pl.DeviceIdType.MESH,
            )

        def recvd(slot, j):
            xfer(slot, slot, j, right).wait_recv()

        sends = []

        def relay(j):
            if j < 4:
                recvd(1, j)
                d = xfer(1, 3, j, right)
            else:
                recvd(2, j)
                d = xfer(2, 3, j, left)
            d.start()
            sends.append(d)

        wqb = wq_ref[...].astype(jnp.bfloat16)

        for b in range(B):
            q_b = jnp.dot(
                x_ref[b].astype(jnp.bfloat16), wqb,
                preferred_element_type=jnp.float32,
            )
            q_b = (q_b * SCALE).astype(jnp.bfloat16)
            for g in range(HKV):
                j = b * HKV + g
                qj = jnp.concatenate(
                    [q_b[:, (g * GQ + t) * DH:(g * GQ + t + 1) * DH]
                     for t in range(GQ)],
                    axis=0,
                )
                kk = k_ref[b, :, g, :].astype(jnp.bfloat16)
                vv = v_ref[b, :, g, :].astype(jnp.bfloat16)
                s = lax.dot_general(
                    qj, kk, (((1,), (1,)), ((), ())),
                    preferred_element_type=jnp.float32,
                )
                p = jnp.exp(s)
                lj = jnp.sum(p, axis=-1, keepdims=True)
                gbuf[0, j, :R, :] = jnp.dot(
                    p.astype(jnp.bfloat16), vv,
                    preferred_element_type=jnp.float32,
                ).astype(jnp.bfloat16)
                gbuf[0, j, R:, :] = lj.astype(jnp.bfloat16).reshape(8, DH)
                for dst_slot, target in ((1, right), (2, left)):
                    d = xfer(0, dst_slot, j, target)
                    d.start()
                    sends.append(d)
                if j >= 2:
                    relay(j - 2)
        relay(6)
        relay(7)

        wob = wo_ref[...].astype(jnp.bfloat16)

        def premerge(j):
            recvd(2 if j < 4 else 1, j)
            return (gbuf[0, j].astype(jnp.float32)
                    + gbuf[1, j].astype(jnp.float32)
                    + gbuf[2, j].astype(jnp.float32))

        def complete(b, qa):
            o2s = []
            for g in range(HKV):
                j = b * HKV + g
                recvd(3, j)
                q4 = qa[j] + gbuf[3, j].astype(jnp.float32)
                den8 = q4[R:, :]
                o3 = q4[:R, :].reshape(8, 128, DH)
                o2s.append((o3 / den8[:, :, None]).reshape(R, DH))
            blocks = []
            for hh in range(HQ):
                g, t = hh // GQ, hh % GQ
                blocks.append(o2s[g][t * SQ:(t + 1) * SQ, :])
            o_b = jnp.concatenate(blocks, axis=1).astype(jnp.bfloat16)
            out_ref[b] = jnp.dot(o_b, wob, preferred_element_type=jnp.float32)

        qa = [None] * NBG
        for j in range(NBG):
            qa[j] = premerge(j)
        for b in range(B):
            complete(b, qa)

        for d in sends:
            d.wait_send()

    return pl.pallas_call(
        body,
        out_shape=jax.ShapeDtypeStruct((B, SQ, D), jnp.float32),
        in_specs=[pl.BlockSpec(memory_space=pltpu.VMEM)] * 5,
        out_specs=pl.BlockSpec(memory_space=pltpu.VMEM),
        scratch_shapes=[
            pltpu.VMEM((N_DEV, NBG, BROWS, DH), jnp.bfloat16),
            pltpu.SemaphoreType.DMA((N_DEV, NBG)),
            pltpu.SemaphoreType.DMA((N_DEV, NBG)),
        ],
        compiler_params=pltpu.CompilerParams(
            collective_id=0,
            vmem_limit_bytes=100 * 1024 * 1024,
        ),
    )(x, Wq, Wo, K_ext, V_ext)


# device time: 58737 ns/iter; 1.0426x vs baseline; 1.0426x over previous
import jax
import jax.numpy as jnp
from jax import lax
from jax.experimental import pallas as pl
from jax.experimental.pallas import tpu as pltpu

N_DEV = 4
B, SQ, HQ, HKV, DH = 4, 256, 8, 2, 128
GQ = HQ // HKV
SKV_LOC = 1024
D = HQ * DH
R = GQ * SQ
NBG = B * HKV
SCALE = 0.08838834764831843

BROWS = R + 8


def kernel(x, Wq, Wo, K_ext, V_ext):

    def body(x_ref, wq_ref, wo_ref, k_ref, v_ref, out_ref,
             gbuf, send_sems, recv_sems):
        my = lax.axis_index("i")
        left = (my + N_DEV - 1) % N_DEV
        right = (my + 1) % N_DEV

        bsem = pltpu.get_barrier_semaphore()
        for nbr in (left, right):
            pl.semaphore_signal(
                bsem, inc=1, device_id=(nbr,),
                device_id_type=pl.DeviceIdType.MESH,
            )
        pl.semaphore_wait(bsem, 2)

        def xfer(src_slot, dst_slot, j, target):
            return pltpu.make_async_remote_copy(
                src_ref=gbuf.at[src_slot, j],
                dst_ref=gbuf.at[dst_slot, j],
                send_sem=send_sems.at[dst_slot, j],
                recv_sem=recv_sems.at[dst_slot, j],
                device_id=(target,), device_id_type=pl.DeviceIdType.MESH,
            )

        def recvd(slot, j):
            xfer(slot, slot, j, right).wait_recv()

        sends = []

        def relay(j):
            if j < 4:
                recvd(1, j)
                d = xfer(1, 3, j, right)
            else:
                recvd(2, j)
                d = xfer(2, 3, j, left)
            d.start()
            sends.append(d)

        wqb = wq_ref[...].astype(jnp.bfloat16)

        for b in range(B):
            q_b = jnp.dot(
                x_ref[b].astype(jnp.bfloat16), wqb,
                preferred_element_type=jnp.float32,
            )
            q_b = (q_b * SCALE).astype(jnp.bfloat16)
            for g in range(HKV):
                j = b * HKV + g
                qj = jnp.concatenate(
                    [q_b[:, (g * GQ + t) * DH:(g * GQ + t + 1) * DH]
                     for t in range(GQ)],
                    axis=0,
                )
                kk = k_ref[b, :, g, :].astype(jnp.bfloat16)
                vv = v_ref[b, :, g, :].astype(jnp.bfloat16)
                s = lax.dot_general(
                    qj, kk, (((1,), (1,)), ((), ())),
                    preferred_element_type=jnp.float32,
                )
                p = jnp.exp(s)
                lj = jnp.sum(p, axis=-1, keepdims=True)
                gbuf[0, j, :R, :] = jnp.dot(
                    p.astype(jnp.bfloat16), vv,
                    preferred_element_type=jnp.float32,
                ).astype(jnp.bfloat16)
                gbuf[0, j, R:, :] = lj.astype(jnp.bfloat16).reshape(8, DH)
                for dst_slot, target in ((1, right), (2, left)):
                    d = xfer(0, dst_slot, j, target)
                    d.start()
                    sends.append(d)
        for j in range(NBG):
            relay(j)

        wob = wo_ref[...].astype(jnp.bfloat16)

        def premerge(j):
            recvd(2 if j < 4 else 1, j)
            return (gbuf[0, j].astype(jnp.float32)
                    + gbuf[1, j].astype(jnp.float32)
                    + gbuf[2, j].astype(jnp.float32))

        def complete(b, qa):
            o2s = []
            for g in range(HKV):
                j = b * HKV + g
                q4 = qa[j] + gbuf[3, j].astype(jnp.float32)
                rden = 1.0 / q4[R:, :]
                o3 = q4[:R, :].reshape(8, 128, DH)
                o2s.append((o3 * rden[:, :, None]).reshape(R, DH))
            blocks = []
            for hh in range(HQ):
                g, t = hh // GQ, hh % GQ
                blocks.append(o2s[g][t * SQ:(t + 1) * SQ, :])
            o_b = jnp.concatenate(blocks, axis=1).astype(jnp.bfloat16)
            out_ref[b] = jnp.dot(o_b, wob, preferred_element_type=jnp.float32)

        qa = [None] * NBG
        for j in range(NBG):
            qa[j] = premerge(j)
        for j in (0, 1, 2, 3):
            recvd(3, j)
        complete(0, qa)
        complete(1, qa)
        for j in (4, 5, 6, 7):
            recvd(3, j)
        complete(2, qa)
        complete(3, qa)

        for d in sends:
            d.wait_send()

    return pl.pallas_call(
        body,
        out_shape=jax.ShapeDtypeStruct((B, SQ, D), jnp.float32),
        in_specs=[pl.BlockSpec(memory_space=pltpu.VMEM)] * 5,
        out_specs=pl.BlockSpec(memory_space=pltpu.VMEM),
        scratch_shapes=[
            pltpu.VMEM((N_DEV, NBG, BROWS, DH), jnp.bfloat16),
            pltpu.SemaphoreType.DMA((N_DEV, NBG)),
            pltpu.SemaphoreType.DMA((N_DEV, NBG)),
        ],
        compiler_params=pltpu.CompilerParams(
            collective_id=0,
            vmem_limit_bytes=100 * 1024 * 1024,
        ),
    )(x, Wq, Wo, K_ext, V_ext)
